# baseline (speedup 1.0000x reference)
import functools

import jax
import jax.numpy as jnp
from jax import lax
from jax.experimental import pallas as pl
from jax.experimental.pallas import tpu as pltpu
from jax.experimental.pallas import tpu_sc as plsc

_VOCAB = 100000
_MAX_POS = 2048
_HIDDEN = 1024
_BATCH = 4
_SEQ = 2048
_EPS = 1e-06

_NC = 2
_NS = 16
_NW = _NC * _NS
_K = 8
_SK = _SEQ // _K
_CB = _BATCH * _SK
_BPW = _CB // _NW
_CH = 16
_NCHUNK = _BPW // _CH

_mesh = plsc.VectorSubcoreMesh(core_axis_name="c", subcore_axis_name="s")


@functools.partial(
    pl.kernel,
    mesh=_mesh,
    out_type=jax.ShapeDtypeStruct((_CB, _HIDDEN), jnp.float32),
    scratch_types=[
        pltpu.VMEM((_NCHUNK, _CH), jnp.int32),
        pltpu.VMEM((_CH, _HIDDEN), jnp.float32),
        pltpu.VMEM((_CH, _HIDDEN), jnp.float32),
        pltpu.SemaphoreType.DMA,
        pltpu.SemaphoreType.DMA,
        pltpu.SemaphoreType.DMA,
        pltpu.SemaphoreType.DMA,
    ],
)
def _sc_gather(idx_hbm, table_hbm, out_hbm,
               idx_v, buf0, buf1, g0, g1, o0, o1):
    wid = lax.axis_index("s") * _NC + lax.axis_index("c")
    base = wid * _BPW
    for c in range(_NCHUNK):
        pltpu.sync_copy(idx_hbm.at[pl.ds(base + c * _CH, _CH)],
                        idx_v.at[c])
    bufs = (buf0, buf1)
    gsems = (g0, g1)
    osems = (o0, o1)
    gathers = [None] * _NCHUNK
    outs = [None] * _NCHUNK
    gathers[0] = pltpu.async_copy(
        table_hbm.at[idx_v.at[0]], bufs[0], gsems[0])
    for c in range(_NCHUNK):
        b = c % 2
        nb = (c + 1) % 2
        if c + 1 < _NCHUNK:
            if c >= 1:
                outs[c - 1].wait()
            gathers[c + 1] = pltpu.async_copy(
                table_hbm.at[idx_v.at[c + 1]], bufs[nb], gsems[nb])
        gathers[c].wait()
        outs[c] = pltpu.async_copy(
            bufs[b], out_hbm.at[pl.ds(base + c * _CH, _CH)], osems[b])
    if _NCHUNK >= 2:
        outs[_NCHUNK - 2].wait()
    outs[_NCHUNK - 1].wait()


def _ln_body_first(x_ref, pos_ref, tok_ref, gamma_ref, beta_ref, out_ref):
    _ln_compute(x_ref, pos_ref, tok_ref, gamma_ref, beta_ref, out_ref)


def _ln_body_chained(x_ref, prev_ref, pos_ref, tok_ref, gamma_ref, beta_ref,
                     out_ref):
    del prev_ref
    _ln_compute(x_ref, pos_ref, tok_ref, gamma_ref, beta_ref, out_ref)


def _ln_compute(x_ref, pos_ref, tok_ref, gamma_ref, beta_ref, out_ref):
    x = x_ref[0] + pos_ref[...] + tok_ref[...]
    mean = jnp.mean(x, axis=-1, keepdims=True)
    xc = x - mean
    var = jnp.mean(xc * xc, axis=-1, keepdims=True)
    out_ref[0] = xc * lax.rsqrt(var + _EPS) * gamma_ref[...] + beta_ref[...]


_LN_ROWS = 256
_JB = _SK // _LN_ROWS


def _tc_layernorm(k, gathered, prev_out, pos_emb, tok_row, gamma, beta):
    grid = (_JB, _BATCH)
    x_spec = pl.BlockSpec((1, _LN_ROWS, _HIDDEN), lambda j, b: (b, j, 0))
    pos_spec = pl.BlockSpec((_LN_ROWS, _HIDDEN),
                            lambda j, b: (k * _JB + j, 0))
    vec_spec = pl.BlockSpec((1, _HIDDEN), lambda j, b: (0, 0))
    out_spec = pl.BlockSpec((1, _LN_ROWS, _HIDDEN),
                            lambda j, b: (b, k * _JB + j, 0))
    if prev_out is None:
        body, in_specs, aliases = _ln_body_first, [x_spec], {}
        args = (gathered,)
    else:
        body = _ln_body_chained
        in_specs = [x_spec, pl.BlockSpec(memory_space=pl.ANY)]
        aliases = {1: 0}
        args = (gathered, prev_out)
    return pl.pallas_call(
        body,
        grid=grid,
        in_specs=in_specs + [pos_spec, vec_spec, vec_spec, vec_spec],
        out_specs=out_spec,
        out_shape=jax.ShapeDtypeStruct((_BATCH, _SEQ, _HIDDEN), jnp.float32),
        input_output_aliases=aliases,
        compiler_params=pltpu.CompilerParams(
            dimension_semantics=("arbitrary", "arbitrary")),
    )(*args, pos_emb, tok_row, gamma, beta)


def kernel(input_ids, word_emb, pos_emb, tok_emb, ln_gamma, ln_beta):
    ids = (input_ids.astype(jnp.int32)
           .reshape(_BATCH, _K, _SK)
           .transpose(1, 0, 2)
           .reshape(_K, _CB))
    tok = tok_emb[0:1]
    gamma = ln_gamma.reshape(1, _HIDDEN)
    beta = ln_beta.reshape(1, _HIDDEN)
    gathers = [
        _sc_gather(ids[k], word_emb).reshape(_BATCH, _SK, _HIDDEN)
        for k in range(_K)
    ]
    out = None
    for k in range(_K):
        out = _tc_layernorm(k, gathers[k], out, pos_emb, tok, gamma, beta)
    return out


# speedup vs baseline: 1.2473x; 1.2473x over previous
import functools

import jax
import jax.numpy as jnp
from jax import lax
from jax.experimental import pallas as pl
from jax.experimental.pallas import tpu as pltpu
from jax.experimental.pallas import tpu_sc as plsc

_VOCAB = 100000
_MAX_POS = 2048
_HIDDEN = 1024
_BATCH = 4
_SEQ = 2048
_EPS = 1e-06

_NC = 2
_NS = 16
_NW = _NC * _NS
_K = 2
_SK = _SEQ // _K
_CB = _BATCH * _SK
_BPW = _CB // _NW
_CH = 32
_NCHUNK = _BPW // _CH

_mesh = plsc.VectorSubcoreMesh(core_axis_name="c", subcore_axis_name="s")


@functools.partial(
    pl.kernel,
    mesh=_mesh,
    out_type=jax.ShapeDtypeStruct((_CB, _HIDDEN), jnp.float32),
    scratch_types=[
        pltpu.VMEM((_NCHUNK, _CH), jnp.int32),
        pltpu.VMEM((_CH, _HIDDEN), jnp.float32),
        pltpu.VMEM((_CH, _HIDDEN), jnp.float32),
        pltpu.SemaphoreType.DMA,
        pltpu.SemaphoreType.DMA,
        pltpu.SemaphoreType.DMA,
        pltpu.SemaphoreType.DMA,
    ],
)
def _sc_gather(idx_hbm, table_hbm, out_hbm,
               idx_v, buf0, buf1, g0, g1, o0, o1):
    wid = lax.axis_index("s") * _NC + lax.axis_index("c")
    base = wid * _BPW
    for c in range(_NCHUNK):
        pltpu.sync_copy(idx_hbm.at[pl.ds(base + c * _CH, _CH)],
                        idx_v.at[c])
    bufs = (buf0, buf1)
    gsems = (g0, g1)
    osems = (o0, o1)
    gathers = [None] * _NCHUNK
    outs = [None] * _NCHUNK
    gathers[0] = pltpu.async_copy(
        table_hbm.at[idx_v.at[0]], bufs[0], gsems[0])
    for c in range(_NCHUNK):
        b = c % 2
        nb = (c + 1) % 2
        if c + 1 < _NCHUNK:
            if c >= 1:
                outs[c - 1].wait()
            gathers[c + 1] = pltpu.async_copy(
                table_hbm.at[idx_v.at[c + 1]], bufs[nb], gsems[nb])
        gathers[c].wait()
        outs[c] = pltpu.async_copy(
            bufs[b], out_hbm.at[pl.ds(base + c * _CH, _CH)], osems[b])
    if _NCHUNK >= 2:
        outs[_NCHUNK - 2].wait()
    outs[_NCHUNK - 1].wait()


def _ln_body_first(x_ref, pos_ref, tok_ref, gamma_ref, beta_ref, out_ref):
    _ln_compute(x_ref, pos_ref, tok_ref, gamma_ref, beta_ref, out_ref)


def _ln_body_chained(x_ref, prev_ref, pos_ref, tok_ref, gamma_ref, beta_ref,
                     out_ref):
    del prev_ref
    _ln_compute(x_ref, pos_ref, tok_ref, gamma_ref, beta_ref, out_ref)


def _ln_compute(x_ref, pos_ref, tok_ref, gamma_ref, beta_ref, out_ref):
    x = x_ref[0] + pos_ref[...] + tok_ref[...]
    mean = jnp.mean(x, axis=-1, keepdims=True)
    xc = x - mean
    var = jnp.mean(xc * xc, axis=-1, keepdims=True)
    out_ref[0] = xc * lax.rsqrt(var + _EPS) * gamma_ref[...] + beta_ref[...]


_LN_ROWS = 512
_JB = _SK // _LN_ROWS


def _tc_layernorm(k, gathered, prev_out, pos_emb, tok_row, gamma, beta):
    grid = (_JB, _BATCH)
    x_spec = pl.BlockSpec((1, _LN_ROWS, _HIDDEN), lambda j, b: (b, j, 0))
    pos_spec = pl.BlockSpec((_LN_ROWS, _HIDDEN),
                            lambda j, b: (k * _JB + j, 0))
    vec_spec = pl.BlockSpec((1, _HIDDEN), lambda j, b: (0, 0))
    out_spec = pl.BlockSpec((1, _LN_ROWS, _HIDDEN),
                            lambda j, b: (b, k * _JB + j, 0))
    if prev_out is None:
        body, in_specs, aliases = _ln_body_first, [x_spec], {}
        args = (gathered,)
    else:
        body = _ln_body_chained
        in_specs = [x_spec, pl.BlockSpec(memory_space=pl.ANY)]
        aliases = {1: 0}
        args = (gathered, prev_out)
    return pl.pallas_call(
        body,
        grid=grid,
        in_specs=in_specs + [pos_spec, vec_spec, vec_spec, vec_spec],
        out_specs=out_spec,
        out_shape=jax.ShapeDtypeStruct((_BATCH, _SEQ, _HIDDEN), jnp.float32),
        input_output_aliases=aliases,
        compiler_params=pltpu.CompilerParams(
            dimension_semantics=("arbitrary", "arbitrary")),
    )(*args, pos_emb, tok_row, gamma, beta)


def kernel(input_ids, word_emb, pos_emb, tok_emb, ln_gamma, ln_beta):
    ids = (input_ids.astype(jnp.int32)
           .reshape(_BATCH, _K, _SK)
           .transpose(1, 0, 2)
           .reshape(_K, _CB))
    tok = tok_emb[0:1]
    gamma = ln_gamma.reshape(1, _HIDDEN)
    beta = ln_beta.reshape(1, _HIDDEN)
    gathers = [
        _sc_gather(ids[k], word_emb).reshape(_BATCH, _SK, _HIDDEN)
        for k in range(_K)
    ]
    out = None
    for k in range(_K):
        out = _tc_layernorm(k, gathers[k], out, pos_emb, tok, gamma, beta)
    return out


# speedup vs baseline: 1.2652x; 1.0144x over previous
import functools

import jax
import jax.numpy as jnp
from jax import lax
from jax.experimental import pallas as pl
from jax.experimental.pallas import tpu as pltpu
from jax.experimental.pallas import tpu_sc as plsc

_VOCAB = 100000
_MAX_POS = 2048
_HIDDEN = 1024
_BATCH = 4
_SEQ = 2048
_EPS = 1e-06

_NC = 2
_NS = 16
_NW = _NC * _NS
_K = 2
_SK = _SEQ // _K
_CB = _BATCH * _SK
_BPW = _CB // _NW
_CH = 32
_NCHUNK = _BPW // _CH

_mesh = plsc.VectorSubcoreMesh(core_axis_name="c", subcore_axis_name="s")


@functools.partial(
    pl.kernel,
    mesh=_mesh,
    out_type=jax.ShapeDtypeStruct((_CB, _HIDDEN), jnp.float32),
    scratch_types=[
        pltpu.VMEM((_NCHUNK, _CH), jnp.int32),
        pltpu.VMEM((_CH, _HIDDEN), jnp.float32),
        pltpu.VMEM((_CH, _HIDDEN), jnp.float32),
        pltpu.SemaphoreType.DMA,
        pltpu.SemaphoreType.DMA,
        pltpu.SemaphoreType.DMA,
        pltpu.SemaphoreType.DMA,
    ],
)
def _sc_gather(idx_hbm, table_hbm, out_hbm,
               idx_v, buf0, buf1, g0, g1, o0, o1):
    wid = lax.axis_index("s") * _NC + lax.axis_index("c")
    base = wid * _BPW
    for c in range(_NCHUNK):
        pltpu.sync_copy(idx_hbm.at[pl.ds(base + c * _CH, _CH)],
                        idx_v.at[c])
    bufs = (buf0, buf1)
    gsems = (g0, g1)
    osems = (o0, o1)
    gathers = [None] * _NCHUNK
    outs = [None] * _NCHUNK
    gathers[0] = pltpu.async_copy(
        table_hbm.at[idx_v.at[0]], bufs[0], gsems[0])
    for c in range(_NCHUNK):
        b = c % 2
        nb = (c + 1) % 2
        if c + 1 < _NCHUNK:
            if c >= 1:
                outs[c - 1].wait()
            gathers[c + 1] = pltpu.async_copy(
                table_hbm.at[idx_v.at[c + 1]], bufs[nb], gsems[nb])
        gathers[c].wait()
        outs[c] = pltpu.async_copy(
            bufs[b], out_hbm.at[pl.ds(base + c * _CH, _CH)], osems[b])
    if _NCHUNK >= 2:
        outs[_NCHUNK - 2].wait()
    outs[_NCHUNK - 1].wait()


def _ln_body_first(x_ref, pos_ref, tok_ref, gamma_ref, beta_ref, out_ref):
    _ln_compute(x_ref, pos_ref, tok_ref, gamma_ref, beta_ref, out_ref)


def _ln_body_chained(x_ref, prev_ref, pos_ref, tok_ref, gamma_ref, beta_ref,
                     out_ref):
    del prev_ref
    _ln_compute(x_ref, pos_ref, tok_ref, gamma_ref, beta_ref, out_ref)


def _ln_compute(x_ref, pos_ref, tok_ref, gamma_ref, beta_ref, out_ref):
    x = x_ref[0] + pos_ref[...] + tok_ref[...]
    mean = jnp.mean(x, axis=-1, keepdims=True)
    xc = x - mean
    var = jnp.mean(xc * xc, axis=-1, keepdims=True)
    out_ref[0] = xc * lax.rsqrt(var + _EPS) * gamma_ref[...] + beta_ref[...]


_LN_ROWS = 1024
_JB = _SK // _LN_ROWS


def _tc_layernorm(k, gathered, prev_out, pos_emb, tok_row, gamma, beta):
    grid = (_JB, _BATCH)
    x_spec = pl.BlockSpec((1, _LN_ROWS, _HIDDEN), lambda j, b: (b, j, 0))
    pos_spec = pl.BlockSpec((_LN_ROWS, _HIDDEN),
                            lambda j, b: (k * _JB + j, 0))
    vec_spec = pl.BlockSpec((1, _HIDDEN), lambda j, b: (0, 0))
    out_spec = pl.BlockSpec((1, _LN_ROWS, _HIDDEN),
                            lambda j, b: (b, k * _JB + j, 0))
    if prev_out is None:
        body, in_specs, aliases = _ln_body_first, [x_spec], {}
        args = (gathered,)
    else:
        body = _ln_body_chained
        in_specs = [x_spec, pl.BlockSpec(memory_space=pl.ANY)]
        aliases = {1: 0}
        args = (gathered, prev_out)
    return pl.pallas_call(
        body,
        grid=grid,
        in_specs=in_specs + [pos_spec, vec_spec, vec_spec, vec_spec],
        out_specs=out_spec,
        out_shape=jax.ShapeDtypeStruct((_BATCH, _SEQ, _HIDDEN), jnp.float32),
        input_output_aliases=aliases,
        compiler_params=pltpu.CompilerParams(
            dimension_semantics=("arbitrary", "arbitrary")),
    )(*args, pos_emb, tok_row, gamma, beta)


def kernel(input_ids, word_emb, pos_emb, tok_emb, ln_gamma, ln_beta):
    ids = (input_ids.astype(jnp.int32)
           .reshape(_BATCH, _K, _SK)
           .transpose(1, 0, 2)
           .reshape(_K, _CB))
    tok = tok_emb[0:1]
    gamma = ln_gamma.reshape(1, _HIDDEN)
    beta = ln_beta.reshape(1, _HIDDEN)
    gathers = [
        _sc_gather(ids[k], word_emb).reshape(_BATCH, _SK, _HIDDEN)
        for k in range(_K)
    ]
    out = None
    for k in range(_K):
        out = _tc_layernorm(k, gathers[k], out, pos_emb, tok, gamma, beta)
    return out
